# Initial kernel scaffold; baseline (speedup 1.0000x reference)
import functools

import jax
import jax.numpy as jnp
from jax import lax
from jax.experimental import pallas as pl
from jax.experimental.pallas import tpu as pltpu
from jax.experimental.pallas import tpu_sc as plsc

N_ROWS = 320000
D = 128
NUM_SEGMENTS = 10000
NC = 2
NS = 16
NW = NC * NS
SEG_PER_W = (NUM_SEGMENTS + NW - 1) // NW
S_PAD = NW * SEG_PER_W
CHUNK = 160
LANES = 16
NVREG = D // LANES
BOUNDS_PAD = 40


def _sc_body(proc_hbm, idx_hbm, bounds_hbm, m_hbm, s_hbm,
             bnd_v, rows_v, idx_v, m_st, s_st):
    cid = lax.axis_index("c")
    sid = lax.axis_index("s")
    wid = sid * NC + cid
    seg_lo = wid * SEG_PER_W

    neg_inf_v = jnp.full((LANES,), -jnp.inf, jnp.float32)
    zero_v = jnp.zeros((LANES,), jnp.float32)

    pltpu.sync_copy(bounds_hbm, bnd_v)
    row_lo = bnd_v[wid]
    row_hi = bnd_v[wid + 1]

    def init_body(i, _):
        for j in range(NVREG):
            m_st[i, pl.ds(j * LANES, LANES)] = neg_inf_v
            s_st[i, pl.ds(j * LANES, LANES)] = zero_v
        return 0
    lax.fori_loop(0, SEG_PER_W, init_body, 0)

    def flush(g_cur, m, s):
        loc = g_cur - seg_lo
        for j in range(NVREG):
            m_st[loc, pl.ds(j * LANES, LANES)] = m[j]
            s_st[loc, pl.ds(j * LANES, LANES)] = s[j]

    def chunk_body(c, carry):
        base = c * CHUNK
        pltpu.sync_copy(proc_hbm.at[pl.ds(base, CHUNK), :], rows_v)
        pltpu.sync_copy(idx_hbm.at[pl.ds(base, CHUNK)], idx_v)
        i_lo = lax.max(row_lo - base, 0)
        i_hi = lax.min(row_hi - base, CHUNK)

        def row_body(i, rcarry):
            g_cur, m, s = rcarry
            g = idx_v[i]
            changed = g != g_cur

            @pl.when(jnp.logical_and(changed, g_cur >= 0))
            def _():
                flush(g_cur, m, s)

            new_m = []
            new_s = []
            for j in range(NVREG):
                mj = jnp.where(changed, neg_inf_v, m[j])
                sj = jnp.where(changed, zero_v, s[j])
                x = rows_v[i, pl.ds(j * LANES, LANES)]
                m2 = jnp.maximum(mj, x)
                s2 = sj * jnp.exp(mj - m2) + jnp.exp(x - m2)
                new_m.append(m2)
                new_s.append(s2)
            return (g, tuple(new_m), tuple(new_s))

        return lax.fori_loop(i_lo, i_hi, row_body, carry)

    c0 = row_lo // CHUNK
    c1 = (row_hi + CHUNK - 1) // CHUNK
    carry0 = (jnp.int32(-1), (neg_inf_v,) * NVREG, (zero_v,) * NVREG)
    g_cur, m, s = lax.fori_loop(c0, c1, chunk_body, carry0)

    @pl.when(g_cur >= 0)
    def _():
        flush(g_cur, m, s)

    pltpu.sync_copy(m_st, m_hbm.at[pl.ds(seg_lo, SEG_PER_W), :])
    pltpu.sync_copy(s_st, s_hbm.at[pl.ds(seg_lo, SEG_PER_W), :])


_sc_call = functools.partial(
    pl.kernel,
    out_type=(
        jax.ShapeDtypeStruct((S_PAD, D), jnp.float32),
        jax.ShapeDtypeStruct((S_PAD, D), jnp.float32),
    ),
    mesh=plsc.VectorSubcoreMesh(
        core_axis_name="c", subcore_axis_name="s",
        num_cores=NC, num_subcores=NS,
    ),
    scratch_types=[
        pltpu.VMEM((BOUNDS_PAD,), jnp.int32),
        pltpu.VMEM((CHUNK, D), jnp.float32),
        pltpu.VMEM((CHUNK,), jnp.int32),
        pltpu.VMEM((SEG_PER_W, D), jnp.float32),
        pltpu.VMEM((SEG_PER_W, D), jnp.float32),
    ],
)(_sc_body)


def _finalize_body(m_ref, s_ref, out_ref):
    m = m_ref[0:NUM_SEGMENTS, :]
    s = s_ref[0:NUM_SEGMENTS, :]
    out = jnp.log(s) + m
    gmax = jnp.max(out)
    t = jnp.sum(jnp.exp(out - gmax))
    z = jnp.log(t) + gmax
    out_ref[...] = out - z


_finalize_call = pl.pallas_call(
    _finalize_body,
    out_shape=jax.ShapeDtypeStruct((NUM_SEGMENTS, D), jnp.float32),
)


@jax.jit
def kernel(proc, idx_b):
    seg_starts = jnp.arange(NW + 1, dtype=jnp.int32) * SEG_PER_W
    bounds = jnp.searchsorted(idx_b, seg_starts, side="left").astype(jnp.int32)
    bounds = jnp.pad(bounds, (0, BOUNDS_PAD - (NW + 1)))
    m_all, s_all = _sc_call(proc, idx_b, bounds)
    return _finalize_call(m_all, s_all)


# speedup vs baseline: 5.5638x; 5.5638x over previous
import functools

import jax
import jax.numpy as jnp
from jax import lax
from jax.experimental import pallas as pl
from jax.experimental.pallas import tpu as pltpu
from jax.experimental.pallas import tpu_sc as plsc

N_ROWS = 320000
D = 128
NUM_SEGMENTS = 10000
NC = 2
NS = 16
NW = NC * NS
SEG_PER_W = 320
S_PAD = NW * SEG_PER_W
CHUNK = 160
LANES = 16
NVREG = D // LANES
BOUNDS_PAD = 48
IDX_PAD = CHUNK + LANES


def _sc_body(proc_hbm, idx_hbm, bounds_hbm, m_hbm, s_hbm,
             bnd_v, rows_v, idx_v, m_st, s_st):
    cid = lax.axis_index("c")
    sid = lax.axis_index("s")
    wid = sid * NC + cid
    seg_lo = pl.multiple_of(wid * SEG_PER_W, 8)

    neg_inf_v = jnp.full((LANES,), -jnp.inf, jnp.float32)
    zero_v = jnp.zeros((LANES,), jnp.float32)

    pltpu.sync_copy(bounds_hbm, bnd_v)
    bnd_vec = bnd_v[pl.ds(wid, LANES)]
    row_lo = bnd_vec[0]
    row_hi = bnd_vec[1]

    def init_body(i, _):
        for j in range(NVREG):
            m_st[pl.ds(i * D + j * LANES, LANES)] = neg_inf_v
            s_st[pl.ds(i * D + j * LANES, LANES)] = zero_v
        return 0
    lax.fori_loop(0, SEG_PER_W, init_body, 0)

    def flush(g_cur, m, s):
        off = (g_cur - seg_lo) * D
        for j in range(NVREG):
            m_st[pl.ds(off + j * LANES, LANES)] = m[j]
            s_st[pl.ds(off + j * LANES, LANES)] = s[j]

    def chunk_body(c, carry):
        base = pl.multiple_of(c * CHUNK, 8)
        pltpu.sync_copy(proc_hbm.at[pl.ds(base * D, CHUNK * D)], rows_v)
        pltpu.sync_copy(idx_hbm.at[pl.ds(base, CHUNK)], idx_v.at[pl.ds(0, CHUNK)])
        i_lo = lax.max(row_lo - base, 0)
        i_hi = lax.min(row_hi - base, CHUNK)

        def row_body(i, rcarry):
            g_cur, m, s = rcarry
            g = idx_v[pl.ds(i, LANES)][0]
            changed = g != g_cur

            @pl.when(jnp.logical_and(changed, g_cur >= 0))
            def _():
                flush(g_cur, m, s)

            new_m = []
            new_s = []
            for j in range(NVREG):
                mj = jnp.where(changed, neg_inf_v, m[j])
                sj = jnp.where(changed, zero_v, s[j])
                x = rows_v[pl.ds(i * D + j * LANES, LANES)]
                m2 = jnp.maximum(mj, x)
                s2 = sj * jnp.exp(mj - m2) + jnp.exp(x - m2)
                new_m.append(m2)
                new_s.append(s2)
            return (g, tuple(new_m), tuple(new_s))

        return lax.fori_loop(i_lo, i_hi, row_body, carry)

    c0 = row_lo // CHUNK
    c1 = (row_hi + CHUNK - 1) // CHUNK
    carry0 = (jnp.int32(-1), (neg_inf_v,) * NVREG, (zero_v,) * NVREG)
    g_cur, m, s = lax.fori_loop(c0, c1, chunk_body, carry0)

    @pl.when(g_cur >= 0)
    def _():
        flush(g_cur, m, s)

    out_off = pl.multiple_of(seg_lo * D, 8)
    pltpu.sync_copy(m_st, m_hbm.at[pl.ds(out_off, SEG_PER_W * D)])
    pltpu.sync_copy(s_st, s_hbm.at[pl.ds(out_off, SEG_PER_W * D)])


_sc_call = functools.partial(
    pl.kernel,
    out_type=(
        jax.ShapeDtypeStruct((S_PAD * D,), jnp.float32),
        jax.ShapeDtypeStruct((S_PAD * D,), jnp.float32),
    ),
    mesh=plsc.VectorSubcoreMesh(
        core_axis_name="c", subcore_axis_name="s",
        num_cores=NC, num_subcores=NS,
    ),
    scratch_types=[
        pltpu.VMEM((BOUNDS_PAD,), jnp.int32),
        pltpu.VMEM((CHUNK * D,), jnp.float32),
        pltpu.VMEM((IDX_PAD,), jnp.int32),
        pltpu.VMEM((SEG_PER_W * D,), jnp.float32),
        pltpu.VMEM((SEG_PER_W * D,), jnp.float32),
    ],
)(_sc_body)


def _finalize_body(m_ref, s_ref, out_ref):
    m = m_ref[0:NUM_SEGMENTS, :]
    s = s_ref[0:NUM_SEGMENTS, :]
    out = jnp.log(s) + m
    gmax = jnp.max(out)
    t = jnp.sum(jnp.exp(out - gmax))
    z = jnp.log(t) + gmax
    out_ref[...] = out - z


_finalize_call = pl.pallas_call(
    _finalize_body,
    out_shape=jax.ShapeDtypeStruct((NUM_SEGMENTS, D), jnp.float32),
)


@jax.jit
def kernel(proc, idx_b):
    seg_starts = jnp.arange(NW + 1, dtype=jnp.int32) * SEG_PER_W
    bounds = jnp.searchsorted(idx_b, seg_starts, side="left").astype(jnp.int32)
    bounds = jnp.pad(bounds, (0, BOUNDS_PAD - (NW + 1)))
    m_all, s_all = _sc_call(proc.reshape(N_ROWS * D), idx_b, bounds)
    return _finalize_call(m_all.reshape(S_PAD, D), s_all.reshape(S_PAD, D))
